# baseline (speedup 1.0000x reference)
import functools

import jax
import jax.numpy as jnp
from jax import lax
from jax.experimental import pallas as pl
from jax.experimental.pallas import tpu as pltpu
from jax.experimental.pallas import tpu_sc as plsc

EMB_D = 64
SEQ_L = 2048
BATCH = 4
TOTAL = BATCH * SEQ_L

NUM_CORES = 2
NUM_SUBCORES = 16
NW = NUM_CORES * NUM_SUBCORES
B_PER_W = TOTAL // NW
IDX_CHUNK = 128
CHUNKS = B_PER_W // IDX_CHUNK
LANES = 16

_mesh = plsc.VectorSubcoreMesh(core_axis_name="c", subcore_axis_name="s")


@functools.partial(
    pl.kernel,
    mesh=_mesh,
    compiler_params=pltpu.CompilerParams(use_tc_tiling_on_sc=False),
    out_type=jax.ShapeDtypeStruct((TOTAL, EMB_D), jnp.float32),
    scratch_types=[
        pltpu.VMEM((CHUNKS, IDX_CHUNK), jnp.int32),
        pltpu.VMEM((B_PER_W, EMB_D), jnp.float32),
        pltpu.VMEM((B_PER_W, EMB_D), jnp.float32),
        pltpu.SemaphoreType.DMA,
    ],
)
def _embed_kernel(idx_hbm, tok_hbm, pos_hbm, out_hbm, idx_v, rows_v, pos_v, sem):
    wid = lax.axis_index("s") * NUM_CORES + lax.axis_index("c")
    base = wid * B_PER_W
    l_start = lax.rem(base, SEQ_L)

    pltpu.sync_copy(idx_hbm.at[pl.ds(wid * CHUNKS, CHUNKS)], idx_v)
    copies = [
        pltpu.async_copy(
            tok_hbm.at[idx_v.at[k]],
            rows_v.at[pl.ds(k * IDX_CHUNK, IDX_CHUNK)],
            sem,
        )
        for k in range(CHUNKS)
    ]
    pltpu.sync_copy(pos_hbm.at[pl.ds(l_start, B_PER_W)], pos_v)
    for cp in copies:
        cp.wait()

    def add_row(r, _):
        for c in range(EMB_D // LANES):
            sl = pl.ds(c * LANES, LANES)
            rows_v[r, sl] = rows_v[r, sl] + pos_v[r, sl]
        return ()

    lax.fori_loop(0, B_PER_W, add_row, ())

    pltpu.sync_copy(rows_v, out_hbm.at[pl.ds(base, B_PER_W)])


def kernel(token_input_ids, tok_table, pos_table):
    idx = token_input_ids.reshape(NW * CHUNKS, IDX_CHUNK).astype(jnp.int32)
    out = _embed_kernel(idx, tok_table, pos_table)
    return out.reshape(BATCH, SEQ_L, EMB_D)


# speedup vs baseline: 1.7045x; 1.7045x over previous
import functools

import jax
import jax.numpy as jnp
from jax import lax
from jax.experimental import pallas as pl
from jax.experimental.pallas import tpu as pltpu
from jax.experimental.pallas import tpu_sc as plsc

EMB_D = 64
SEQ_L = 2048
BATCH = 4
TOTAL = BATCH * SEQ_L

NUM_CORES = 2
NUM_SUBCORES = 16
NW = NUM_CORES * NUM_SUBCORES
B_PER_W = TOTAL // NW
LANES = 16

_mesh = plsc.VectorSubcoreMesh(core_axis_name="c", subcore_axis_name="s")


@functools.partial(
    pl.kernel,
    mesh=_mesh,
    compiler_params=pltpu.CompilerParams(needs_layout_passes=False),
    out_type=jax.ShapeDtypeStruct((TOTAL, EMB_D), jnp.float32),
    scratch_types=[
        pltpu.VMEM((B_PER_W,), jnp.int32),
        pltpu.VMEM((B_PER_W, EMB_D), jnp.float32),
        pltpu.VMEM((B_PER_W, EMB_D), jnp.float32),
        pltpu.SemaphoreType.DMA,
    ],
)
def _embed_kernel(idx_hbm, tok_hbm, pos_hbm, out_hbm,
                  idx_v, rows_v, pos_v, sem):
    wid = lax.axis_index("s") * NUM_CORES + lax.axis_index("c")
    base = wid * B_PER_W
    l_start = lax.rem(base, SEQ_L)

    pltpu.sync_copy(idx_hbm.at[pl.ds(base, B_PER_W)], idx_v)

    iota = lax.iota(jnp.int32, LANES)

    def fire(g, _):
        idv = idx_v[pl.ds(g * LANES, LANES)]
        for j in range(LANES):
            s = jnp.sum(jnp.where(iota == j, idv, 0))
            pltpu.async_copy(tok_hbm.at[s], rows_v.at[g * LANES + j], sem)
        return ()

    lax.fori_loop(0, B_PER_W // LANES, fire, ())
    pltpu.sync_copy(pos_hbm.at[pl.ds(l_start, B_PER_W)], pos_v)
    pltpu.make_async_copy(tok_hbm.at[pl.ds(0, B_PER_W)], rows_v, sem).wait()

    def add_row(r, _):
        for c in range(EMB_D // LANES):
            sl = pl.ds(c * LANES, LANES)
            rows_v[r, sl] = rows_v[r, sl] + pos_v[r, sl]
        return ()

    lax.fori_loop(0, B_PER_W, add_row, ())

    pltpu.sync_copy(rows_v, out_hbm.at[pl.ds(base, B_PER_W)])


def kernel(token_input_ids, tok_table, pos_table):
    idx = token_input_ids.reshape(TOTAL).astype(jnp.int32)
    out = _embed_kernel(idx, tok_table, pos_table)
    return out.reshape(BATCH, SEQ_L, EMB_D)
